# baseline (speedup 1.0000x reference)
import functools

import jax
import jax.numpy as jnp
import numpy as np
from jax.experimental import pallas as pl
from jax.experimental.pallas import tpu as pltpu

_SA_CFG = [
    (1024, [0.05, 0.1], [16, 32], 3, [[16, 16, 32], [32, 32, 64]]),
    (256, [0.1, 0.2], [16, 32], 96, [[64, 64, 128], [64, 96, 128]]),
    (64, [0.2, 0.4], [16, 32], 256, [[128, 196, 256], [128, 192, 256]]),
    (16, [0.4, 0.8], [16, 32], 512, [[256, 256, 512], [256, 384, 512]]),
    (8, [0.8, 1.0], [16, 16], 1024, [[256, 256, 512], [256, 384, 512]]),
]
_EPS = 1e-5


def _square_distance(src, dst):
    dist = -2.0 * jnp.matmul(src, dst.transpose(0, 2, 1))
    dist = dist + jnp.sum(src ** 2, -1)[:, :, None]
    dist = dist + jnp.sum(dst ** 2, -1)[:, None, :]
    return dist


def _index_points(points, idx):
    B = points.shape[0]
    batch = jnp.arange(B).reshape((B,) + (1,) * (idx.ndim - 1))
    return points[batch, idx]


def _farthest_point_sample(xyz, npoint):
    xyz = jax.lax.stop_gradient(xyz)
    B, N, _ = xyz.shape
    def body(i, state):
        centroids, distance, farthest = state
        centroids = centroids.at[:, i].set(farthest)
        centroid = xyz[jnp.arange(B), farthest][:, None, :]
        dist = jnp.sum((xyz - centroid) ** 2, -1)
        distance = jnp.minimum(distance, dist)
        farthest = jnp.argmax(distance, -1).astype(jnp.int32)
        return centroids, distance, farthest
    init = (jnp.zeros((B, npoint), jnp.int32), jnp.full((B, N), 1e10, jnp.float32), jnp.zeros((B,), jnp.int32))
    centroids, _, _ = jax.lax.fori_loop(0, npoint, body, init)
    return centroids


def _query_ball_point(radius, nsample, xyz, new_xyz):
    B, N, _ = xyz.shape
    S = new_xyz.shape[1]
    sqrdists = jax.lax.stop_gradient(_square_distance(new_xyz, xyz))
    group_idx = jnp.broadcast_to(jnp.arange(N, dtype=jnp.int32), (B, S, N))
    group_idx = jnp.where(sqrdists > radius ** 2, N, group_idx)
    group_idx = jnp.sort(group_idx, axis=-1)[:, :, :nsample]
    group_first = jnp.broadcast_to(group_idx[:, :, :1], group_idx.shape)
    group_idx = jnp.where(group_idx == N, group_first, group_idx)
    return group_idx


def _conv_bn_relu_2d(p, x):
    x = jnp.einsum('od,bdks->boks', p['W'], x) + p['b'][None, :, None, None]
    mean = jnp.mean(x, axis=(0, 2, 3), keepdims=True)
    var = jnp.var(x, axis=(0, 2, 3), keepdims=True)
    x = (x - mean) / jnp.sqrt(var + _EPS)
    x = x * p['gamma'][None, :, None, None] + p['beta'][None, :, None, None]
    return jax.nn.relu(x)


def _conv_bn_relu_1d(p, x):
    x = jnp.einsum('od,bdn->bon', p['W'], x) + p['b'][None, :, None]
    mean = jnp.mean(x, axis=(0, 2), keepdims=True)
    var = jnp.var(x, axis=(0, 2), keepdims=True)
    x = (x - mean) / jnp.sqrt(var + _EPS)
    x = x * p['gamma'][None, :, None] + p['beta'][None, :, None]
    return jax.nn.relu(x)


def _sa_msg(cfg, branch_params, xyz, points):
    npoint, radii, nsamples, _, _ = cfg
    xyz_t = xyz.transpose(0, 2, 1)
    pts_t = points.transpose(0, 2, 1) if points is not None else None
    fps_idx = _farthest_point_sample(xyz_t, npoint)
    new_xyz = _index_points(xyz_t, fps_idx)
    outs = []
    for i, radius in enumerate(radii):
        K = nsamples[i]
        group_idx = _query_ball_point(radius, K, xyz_t, new_xyz)
        grouped_xyz = _index_points(xyz_t, group_idx) - new_xyz[:, :, None, :]
        if pts_t is not None:
            grouped_points = jnp.concatenate([_index_points(pts_t, group_idx), grouped_xyz], axis=-1)
        else:
            grouped_points = grouped_xyz
        g = grouped_points.transpose(0, 3, 2, 1)
        for p in branch_params[i]:
            g = _conv_bn_relu_2d(p, g)
        outs.append(jnp.max(g, axis=2))
    return new_xyz.transpose(0, 2, 1), jnp.concatenate(outs, axis=1)


def _fp_layer(blk, xyz1, xyz2, points1, points2):
    xyz1_t = xyz1.transpose(0, 2, 1)
    xyz2_t = xyz2.transpose(0, 2, 1)
    pts2 = points2.transpose(0, 2, 1)
    N = xyz1_t.shape[1]
    S = xyz2_t.shape[1]
    if S == 1:
        interpolated = jnp.repeat(pts2, N, axis=1)
    else:
        dists = _square_distance(xyz1_t, xyz2_t)
        idx = jnp.argsort(dists, axis=-1)[:, :, :3]
        d3 = jnp.take_along_axis(dists, idx, axis=-1)
        dist_recip = 1.0 / (d3 + 1e-8)
        norm = jnp.sum(dist_recip, axis=2, keepdims=True)
        weight = dist_recip / norm
        interpolated = jnp.sum(_index_points(pts2, idx) * weight[..., None], axis=2)
    if points1 is not None:
        new_points = jnp.concatenate([points1.transpose(0, 2, 1), interpolated], axis=-1)
    else:
        new_points = interpolated
    x = new_points.transpose(0, 2, 1)
    for p in blk:
        x = _conv_bn_relu_1d(p, x)
    return x


def _head_kernel(x_ref, w1_ref, b1_ref, g1_ref, be1_ref, w2_ref, b2_ref, out_ref):
    B, C, N = x_ref.shape
    x = x_ref[...].reshape(B * C, N).reshape(B, C, N)
    x2 = jnp.transpose(x, (1, 0, 2)).reshape(C, B * N)
    w1 = w1_ref[...]
    y = jax.lax.dot_general(w1, x2, (((1,), (0,)), ((), ())),
                            preferred_element_type=jnp.float32,
                            precision=jax.lax.Precision.HIGHEST)
    y = y + b1_ref[...].reshape(C, 1)
    m = jnp.mean(y, axis=1, keepdims=True)
    v = jnp.mean((y - m) ** 2, axis=1, keepdims=True)
    y = (y - m) / jnp.sqrt(v + _EPS)
    y = y * g1_ref[...].reshape(C, 1) + be1_ref[...].reshape(C, 1)
    y = jnp.maximum(y, 0.0)
    w2 = w2_ref[...]
    z = jax.lax.dot_general(w2, y, (((1,), (0,)), ((), ())),
                            preferred_element_type=jnp.float32,
                            precision=jax.lax.Precision.HIGHEST)
    z = z + b2_ref[...].reshape(-1, 1)
    z = jax.nn.sigmoid(z)
    zt = jnp.transpose(z.reshape(z.shape[0], B, N), (1, 2, 0))
    out_ref[...] = zt


def _head(params, x):
    B, C, N = x.shape
    h1 = params['head1']
    h2 = params['head2']
    out = pl.pallas_call(
        _head_kernel,
        out_shape=jax.ShapeDtypeStruct((B, N, h2['W'].shape[0]), jnp.float32),
    )(x, h1['W'], h1['b'], h1['gamma'], h1['beta'], h2['W'], h2['b'])
    return out


def kernel(xyz, params):
    l0_points = xyz
    l0_xyz = xyz[:, :3, :]
    l1_xyz, l1_points = _sa_msg(_SA_CFG[0], params['sa1'], l0_xyz, l0_points)
    l2_xyz, l2_points = _sa_msg(_SA_CFG[1], params['sa2'], l1_xyz, l1_points)
    l3_xyz, l3_points = _sa_msg(_SA_CFG[2], params['sa3'], l2_xyz, l2_points)
    l4_xyz, l4_points = _sa_msg(_SA_CFG[3], params['sa4'], l3_xyz, l3_points)
    l5_xyz, l5_points = _sa_msg(_SA_CFG[4], params['sa5'], l4_xyz, l4_points)
    l4_points = _fp_layer(params['fp5'], l4_xyz, l5_xyz, l4_points, l5_points)
    l3_points = _fp_layer(params['fp4'], l3_xyz, l4_xyz, l3_points, l4_points)
    l2_points = _fp_layer(params['fp3'], l2_xyz, l3_xyz, l2_points, l3_points)
    l1_points = _fp_layer(params['fp2'], l1_xyz, l2_xyz, l1_points, l2_points)
    l0_points = _fp_layer(params['fp1'], l0_xyz, l1_xyz, None, l1_points)
    return _head(params, l0_points)


# speedup vs baseline: 3.2100x; 3.2100x over previous
import functools

import jax
import jax.numpy as jnp
import numpy as np
from jax.experimental import pallas as pl
from jax.experimental.pallas import tpu as pltpu

_SA_CFG = [
    (1024, [0.05, 0.1], [16, 32], 3, [[16, 16, 32], [32, 32, 64]]),
    (256, [0.1, 0.2], [16, 32], 96, [[64, 64, 128], [64, 96, 128]]),
    (64, [0.2, 0.4], [16, 32], 256, [[128, 196, 256], [128, 192, 256]]),
    (16, [0.4, 0.8], [16, 32], 512, [[256, 256, 512], [256, 384, 512]]),
    (8, [0.8, 1.0], [16, 16], 1024, [[256, 256, 512], [256, 384, 512]]),
]
_EPS = 1e-5


def _index_points(points, idx):
    B = points.shape[0]
    batch = jnp.arange(B).reshape((B,) + (1,) * (idx.ndim - 1))
    return points[batch, idx]


def _fps_kernel(npoint, xyz_ref, out_ref):
    B, _, N = xyz_ref.shape
    xyz = xyz_ref[...]
    iota = jax.lax.broadcasted_iota(jnp.int32, (B, N), 1)

    def body(i, carry):
        distance, farthest = carry
        out_ref[pl.ds(i, 1), :] = farthest.reshape(1, B)
        oh = (iota == farthest[:, None]).astype(jnp.float32)
        centroid = jnp.sum(xyz * oh[:, None, :], axis=2)
        diff = xyz - centroid[:, :, None]
        dist = jnp.sum(diff * diff, axis=1)
        distance = jnp.minimum(distance, dist)
        m = jnp.max(distance, axis=1, keepdims=True)
        farthest = jnp.min(jnp.where(distance == m, iota, N), axis=1)
        return distance, farthest

    init = (jnp.full((B, N), 1e10, jnp.float32), jnp.zeros((B,), jnp.int32))
    jax.lax.fori_loop(0, npoint, body, init)


def _fps(xyz, npoint):
    B, _, N = xyz.shape
    out = pl.pallas_call(
        functools.partial(_fps_kernel, npoint),
        out_shape=jax.ShapeDtypeStruct((npoint, B), jnp.int32),
    )(xyz)
    return out.T


def _sqdist(q, x):
    mm = jax.lax.dot_general(
        q.astype(jnp.bfloat16), x.astype(jnp.bfloat16),
        (((0,), (0,)), ((), ())),
        preferred_element_type=jnp.float32)
    qsq = (q[0] * q[0] + q[1] * q[1]) + q[2] * q[2]
    xsq = (x[0] * x[0] + x[1] * x[1]) + x[2] * x[2]
    return (-2.0 * mm + qsq[:, None]) + xsq[None, :]


def _ball_kernel(radius2, K, N, q_ref, x_ref, out_ref):
    q = q_ref[0]
    x = x_ref[0]
    Sc = q.shape[1]
    d2 = _sqdist(q, x)
    iota = jax.lax.broadcasted_iota(jnp.int32, (Sc, N), 1)
    cand = jnp.where(d2 <= radius2, iota, N)
    first = None
    for k in range(K):
        m = jnp.min(cand, axis=1)
        if first is None:
            first = m
            sel = m
        else:
            sel = jnp.where(m == N, first, m)
        out_ref[0, k, :] = sel
        cand = jnp.where(iota == m[:, None], N, cand)


def _query_ball_point(radius, K, xyz, new_xyz):
    B, _, N = xyz.shape
    S = new_xyz.shape[2]
    Sc = min(S, 256)
    grid = (B, S // Sc)
    out = pl.pallas_call(
        functools.partial(_ball_kernel, np.float32(radius ** 2), K, N),
        grid=grid,
        in_specs=[
            pl.BlockSpec((1, 3, Sc), lambda b, s: (b, 0, s)),
            pl.BlockSpec((1, 3, N), lambda b, s: (b, 0, 0)),
        ],
        out_specs=pl.BlockSpec((1, K, Sc), lambda b, s: (b, 0, s)),
        out_shape=jax.ShapeDtypeStruct((B, K, S), jnp.int32),
    )(new_xyz, xyz)
    return out.transpose(0, 2, 1)


def _knn3_kernel(S, q_ref, s_ref, idx_ref, d3_ref):
    q = q_ref[0]
    src = s_ref[0]
    Qc = q.shape[1]
    d2 = _sqdist(q, src)
    iota = jax.lax.broadcasted_iota(jnp.int32, (Qc, S), 1)
    work = d2
    for j in range(3):
        m = jnp.min(work, axis=1, keepdims=True)
        ij = jnp.min(jnp.where(work == m, iota, S), axis=1)
        idx_ref[0, j, :] = ij
        d3_ref[0, j, :] = m[:, 0]
        work = jnp.where(iota == ij[:, None], jnp.float32(3.4e38), work)


def _wgather_kernel(S, p_ref, idx_ref, w_ref, out_ref):
    pts = p_ref[0]
    idx = idx_ref[0]
    w = w_ref[0]
    Qc = idx.shape[1]
    iota = jax.lax.broadcasted_iota(jnp.int32, (Qc, S), 1)
    out = None
    for j in range(3):
        onehot = (iota == idx[j][:, None]).astype(jnp.float32)
        g = jax.lax.dot_general(
            pts, onehot, (((1,), (1,)), ((), ())),
            preferred_element_type=jnp.float32,
            precision=jax.lax.Precision.HIGHEST)
        term = g * w[j][None, :]
        out = term if out is None else out + term
    out_ref[0] = out


def _three_interp(xyz1, xyz2, pts2):
    B, _, Nq = xyz1.shape
    S = xyz2.shape[2]
    C = pts2.shape[1]
    Qc = min(Nq, 1024)
    grid = (B, Nq // Qc)
    idx, d3 = pl.pallas_call(
        functools.partial(_knn3_kernel, S),
        grid=grid,
        in_specs=[
            pl.BlockSpec((1, 3, Qc), lambda b, s: (b, 0, s)),
            pl.BlockSpec((1, 3, S), lambda b, s: (b, 0, 0)),
        ],
        out_specs=[
            pl.BlockSpec((1, 3, Qc), lambda b, s: (b, 0, s)),
            pl.BlockSpec((1, 3, Qc), lambda b, s: (b, 0, s)),
        ],
        out_shape=[
            jax.ShapeDtypeStruct((B, 3, Nq), jnp.int32),
            jax.ShapeDtypeStruct((B, 3, Nq), jnp.float32),
        ],
    )(xyz1, xyz2)
    recip = 1.0 / (d3.transpose(0, 2, 1) + 1e-8)
    norm = jnp.sum(recip, axis=2, keepdims=True)
    w = (recip / norm).transpose(0, 2, 1)
    out = pl.pallas_call(
        functools.partial(_wgather_kernel, S),
        grid=grid,
        in_specs=[
            pl.BlockSpec((1, C, S), lambda b, s: (b, 0, 0)),
            pl.BlockSpec((1, 3, Qc), lambda b, s: (b, 0, s)),
            pl.BlockSpec((1, 3, Qc), lambda b, s: (b, 0, s)),
        ],
        out_specs=pl.BlockSpec((1, C, Qc), lambda b, s: (b, 0, s)),
        out_shape=jax.ShapeDtypeStruct((B, C, Nq), jnp.float32),
    )(pts2, idx, w)
    return out


def _conv_bn_relu_2d(p, x):
    x = jnp.einsum('od,bdks->boks', p['W'], x) + p['b'][None, :, None, None]
    mean = jnp.mean(x, axis=(0, 2, 3), keepdims=True)
    var = jnp.var(x, axis=(0, 2, 3), keepdims=True)
    x = (x - mean) / jnp.sqrt(var + _EPS)
    x = x * p['gamma'][None, :, None, None] + p['beta'][None, :, None, None]
    return jax.nn.relu(x)


def _conv_bn_relu_1d(p, x):
    x = jnp.einsum('od,bdn->bon', p['W'], x) + p['b'][None, :, None]
    mean = jnp.mean(x, axis=(0, 2), keepdims=True)
    var = jnp.var(x, axis=(0, 2), keepdims=True)
    x = (x - mean) / jnp.sqrt(var + _EPS)
    x = x * p['gamma'][None, :, None] + p['beta'][None, :, None]
    return jax.nn.relu(x)


def _sa_msg(cfg, branch_params, xyz, points):
    npoint, radii, nsamples, _, _ = cfg
    fps_idx = _fps(xyz, npoint)
    xyz_t = xyz.transpose(0, 2, 1)
    pts_t = points.transpose(0, 2, 1) if points is not None else None
    new_xyz = _index_points(xyz_t, fps_idx)
    new_xyz_c = new_xyz.transpose(0, 2, 1)
    outs = []
    for i, radius in enumerate(radii):
        K = nsamples[i]
        group_idx = _query_ball_point(radius, K, xyz, new_xyz_c)
        grouped_xyz = _index_points(xyz_t, group_idx) - new_xyz[:, :, None, :]
        if pts_t is not None:
            grouped_points = jnp.concatenate(
                [_index_points(pts_t, group_idx), grouped_xyz], axis=-1)
        else:
            grouped_points = grouped_xyz
        g = grouped_points.transpose(0, 3, 2, 1)
        for p in branch_params[i]:
            g = _conv_bn_relu_2d(p, g)
        outs.append(jnp.max(g, axis=2))
    return new_xyz_c, jnp.concatenate(outs, axis=1)


def _fp_layer(blk, xyz1, xyz2, points1, points2):
    N = xyz1.shape[2]
    S = xyz2.shape[2]
    if S == 1:
        interpolated = jnp.repeat(points2, N, axis=2)
    else:
        interpolated = _three_interp(xyz1, xyz2, points2)
    if points1 is not None:
        x = jnp.concatenate([points1, interpolated], axis=1)
    else:
        x = interpolated
    for p in blk:
        x = _conv_bn_relu_1d(p, x)
    return x


def _head_kernel(x_ref, w1_ref, b1_ref, g1_ref, be1_ref, w2_ref, b2_ref, out_ref):
    B, C, N = x_ref.shape
    x = x_ref[...].reshape(B * C, N).reshape(B, C, N)
    x2 = jnp.transpose(x, (1, 0, 2)).reshape(C, B * N)
    w1 = w1_ref[...]
    y = jax.lax.dot_general(w1, x2, (((1,), (0,)), ((), ())),
                            preferred_element_type=jnp.float32,
                            precision=jax.lax.Precision.HIGHEST)
    y = y + b1_ref[...].reshape(C, 1)
    m = jnp.mean(y, axis=1, keepdims=True)
    v = jnp.mean((y - m) ** 2, axis=1, keepdims=True)
    y = (y - m) / jnp.sqrt(v + _EPS)
    y = y * g1_ref[...].reshape(C, 1) + be1_ref[...].reshape(C, 1)
    y = jnp.maximum(y, 0.0)
    w2 = w2_ref[...]
    z = jax.lax.dot_general(w2, y, (((1,), (0,)), ((), ())),
                            preferred_element_type=jnp.float32,
                            precision=jax.lax.Precision.HIGHEST)
    z = z + b2_ref[...].reshape(-1, 1)
    z = jax.nn.sigmoid(z)
    zt = jnp.transpose(z.reshape(z.shape[0], B, N), (1, 2, 0))
    out_ref[...] = zt


def _head(params, x):
    B, C, N = x.shape
    h1 = params['head1']
    h2 = params['head2']
    out = pl.pallas_call(
        _head_kernel,
        out_shape=jax.ShapeDtypeStruct((B, N, h2['W'].shape[0]), jnp.float32),
    )(x, h1['W'], h1['b'], h1['gamma'], h1['beta'], h2['W'], h2['b'])
    return out


def kernel(xyz, params):
    l0_points = xyz
    l0_xyz = xyz[:, :3, :]
    l1_xyz, l1_points = _sa_msg(_SA_CFG[0], params['sa1'], l0_xyz, l0_points)
    l2_xyz, l2_points = _sa_msg(_SA_CFG[1], params['sa2'], l1_xyz, l1_points)
    l3_xyz, l3_points = _sa_msg(_SA_CFG[2], params['sa3'], l2_xyz, l2_points)
    l4_xyz, l4_points = _sa_msg(_SA_CFG[3], params['sa4'], l3_xyz, l3_points)
    l5_xyz, l5_points = _sa_msg(_SA_CFG[4], params['sa5'], l4_xyz, l4_points)
    l4_points = _fp_layer(params['fp5'], l4_xyz, l5_xyz, l4_points, l5_points)
    l3_points = _fp_layer(params['fp4'], l3_xyz, l4_xyz, l3_points, l4_points)
    l2_points = _fp_layer(params['fp3'], l2_xyz, l3_xyz, l2_points, l3_points)
    l1_points = _fp_layer(params['fp2'], l1_xyz, l2_xyz, l1_points, l2_points)
    l0_points = _fp_layer(params['fp1'], l0_xyz, l1_xyz, None, l1_points)
    return _head(params, l0_points)
